# baseline (speedup 1.0000x reference)
import dataclasses

import jax
import jax.numpy as jnp
from jax.experimental import pallas as pl
from jax.experimental.pallas import tpu as pltpu
from jax.experimental.pallas import tpu_sc as plsc

_VOCAB = 2048
_ROWS_PER_TILE = 512
_W = 16
_NBLK = _ROWS_PER_TILE // _W
_BUFLEN = _W * _VOCAB


def _body(ids_hbm, out_hbm, ids_vmem, bufs, sems, ids_sem):
    core = jax.lax.axis_index("c")
    sub = jax.lax.axis_index("s")
    gid = core * 16 + sub
    row0 = gid * _ROWS_PER_TILE

    pltpu.make_async_copy(
        ids_hbm.at[pl.ds(row0, _ROWS_PER_TILE)], ids_vmem, ids_sem
    ).start()

    minus50 = jnp.full((16,), -50.0, jnp.float32)
    fifty = jnp.full((16,), 50.0, jnp.float32)
    iota = jax.lax.iota(jnp.int32, 16)
    row_off = iota * _VOCAB

    @pl.loop(0, 2 * _BUFLEN // 16)
    def _(c):
        bufs[pl.ds(c * 16, 16)] = minus50

    pltpu.make_async_copy(
        ids_hbm.at[pl.ds(row0, _ROWS_PER_TILE)], ids_vmem, ids_sem
    ).wait()

    out_base = row0 * _VOCAB
    for b in range(_NBLK):
        slot = b % 2
        slot_base = slot * _BUFLEN
        if b >= 2:
            pltpu.make_async_copy(
                bufs.at[pl.ds(slot_base, _BUFLEN)],
                out_hbm.at[pl.ds(out_base + (b - 2) * _BUFLEN, _BUFLEN)],
                sems.at[slot],
            ).wait()
            old_ids = ids_vmem[pl.ds((b - 2) * _W, 16)]
            old_addr = slot_base + row_off + old_ids % 3 + 1
            plsc.store_scatter(bufs, [old_addr], minus50)
        ids_v = ids_vmem[pl.ds(b * _W, 16)]
        addr = slot_base + row_off + ids_v % 3 + 1
        plsc.store_scatter(bufs, [addr], fifty)
        pltpu.make_async_copy(
            bufs.at[pl.ds(slot_base, _BUFLEN)],
            out_hbm.at[pl.ds(out_base + b * _BUFLEN, _BUFLEN)],
            sems.at[slot],
        ).start()

    for b in (_NBLK - 2, _NBLK - 1):
        slot = b % 2
        pltpu.make_async_copy(
            bufs.at[pl.ds(slot * _BUFLEN, _BUFLEN)],
            out_hbm.at[pl.ds(out_base + b * _BUFLEN, _BUFLEN)],
            sems.at[slot],
        ).wait()


def kernel(input_ids):
    b, s = input_ids.shape
    n = b * s
    ids_flat = input_ids.reshape(n)

    cp = pltpu.CompilerParams()
    if "needs_layout_passes" in pltpu.CompilerParams.__dataclass_fields__:
        cp = dataclasses.replace(cp, needs_layout_passes=False)
    f = pl.kernel(
        _body,
        out_type=jax.ShapeDtypeStruct((n * _VOCAB,), jnp.float32),
        mesh=plsc.VectorSubcoreMesh(core_axis_name="c", subcore_axis_name="s"),
        compiler_params=cp,
        scratch_types=[
            pltpu.VMEM((_ROWS_PER_TILE,), jnp.int32),
            pltpu.VMEM((2 * _BUFLEN,), jnp.float32),
            pltpu.SemaphoreType.DMA((2,)),
            pltpu.SemaphoreType.DMA,
        ],
    )
    out = f(ids_flat)
    return out.reshape(b, s, _VOCAB)


# speedup vs baseline: 3.8382x; 3.8382x over previous
import jax
import jax.numpy as jnp
from jax.experimental import pallas as pl

_VOCAB = 2048
_SBLK = 1024


def _body(ids_ref, out_ref):
    ids = ids_ref[0]
    pred = ids % 3 + 1
    iota = jax.lax.broadcasted_iota(jnp.int32, (_SBLK, _VOCAB), 1)
    out_ref[0] = jnp.where(iota == pred, 50.0, -50.0)


def kernel(input_ids):
    b, s = input_ids.shape
    n = b * s
    nblk = n // _SBLK
    ids3 = input_ids.reshape(nblk, _SBLK, 1)
    out = pl.pallas_call(
        _body,
        grid=(nblk,),
        in_specs=[pl.BlockSpec((1, _SBLK, 1), lambda i: (i, 0, 0))],
        out_specs=pl.BlockSpec((1, _SBLK, _VOCAB), lambda i: (i, 0, 0)),
        out_shape=jax.ShapeDtypeStruct((nblk, _SBLK, _VOCAB), jnp.float32),
    )(ids3)
    return out.reshape(b, s, _VOCAB)
